# baseline (speedup 1.0000x reference)
import functools

import jax
import jax.numpy as jnp
from jax import lax
from jax.experimental import pallas as pl
from jax.experimental.pallas import tpu as pltpu
from jax.experimental.pallas import tpu_sc as plsc


def _proj_body(tblT_ref, w_ref, bias_ref, out_ref):
    r = lax.dot_general(
        w_ref[...], tblT_ref[...],
        (((1,), (0,)), ((), ())),
        preferred_element_type=jnp.float32,
    )
    out_ref[...] = r[0] + bias_ref[0, 0]


def _project_table(tableT, w_scaled, bias_scaled, blk=65536):
    D, V = tableT.shape
    grid = pl.cdiv(V, blk)
    return pl.pallas_call(
        _proj_body,
        grid=(grid,),
        in_specs=[
            pl.BlockSpec((D, blk), lambda i: (0, i)),
            pl.BlockSpec((1, D), lambda i: (0, 0)),
            pl.BlockSpec(memory_space=pltpu.SMEM),
        ],
        out_specs=pl.BlockSpec((blk,), lambda i: (i,)),
        out_shape=jax.ShapeDtypeStruct((V,), jnp.float32),
    )(tableT, w_scaled, bias_scaled)


def _make_sc_gather(V, S, B, NC, NS, C):
    NW = NC * NS
    per_w = B // NW
    n_chunks = per_w // C
    n_seg = C // 16

    mesh = plsc.VectorSubcoreMesh(core_axis_name="c", subcore_axis_name="s")

    S_LO = (S * 5 // 8) // 2 * 2
    S_HI = S - S_LO

    @functools.partial(
        pl.kernel,
        mesh=mesh,
        out_type=jax.ShapeDtypeStruct((B,), jnp.float32),
        scratch_types=[
            pltpu.VMEM((S_LO * C,), jnp.int32),
            pltpu.VMEM((S_HI * C,), jnp.int32),
            pltpu.VMEM((S_LO * C,), jnp.int32),
            pltpu.VMEM((S_HI * C,), jnp.int32),
            pltpu.VMEM((S_LO * C,), jnp.float32),
            pltpu.VMEM((S_HI * C,), jnp.float32),
            pltpu.VMEM((S_LO * C,), jnp.float32),
            pltpu.VMEM((S_HI * C,), jnp.float32),
            pltpu.VMEM((C,), jnp.float32),
            pltpu.VMEM_SHARED((V,), jnp.float32),
            pltpu.SemaphoreType.DMA,
            pltpu.SemaphoreType.DMA,
            pltpu.SemaphoreType.DMA,
            pltpu.SemaphoreType.DMA,
            pltpu.SemaphoreType.DMA,
            pltpu.SemaphoreType.DMA,
        ],
    )
    def sc_gather(v_hbm, xp_hbm, out_hbm,
                  il0, ih0, il1, ih1, vl0, vh0, vl1, vh1,
                  outb_v, v_sh, ssem0, ssem1, gl0, gl1, gh0, gh1):
        sid = lax.axis_index("s")
        wid = sid * NC + lax.axis_index("c")
        idx_lo = (il0, il1)
        idx_hi = (ih0, ih1)
        vals_lo = (vl0, vl1)
        vals_hi = (vh0, vh1)
        ssems = (ssem0, ssem1)
        gsems_lo = (gl0, gl1)
        gsems_hi = (gh0, gh1)

        def stage_start(ci, buf):
            base = (wid * per_w + ci * C) * S
            lo = pltpu.async_copy(
                xp_hbm.at[pl.ds(base, S_LO * C)], idx_lo[buf], ssems[buf])
            hi = pltpu.async_copy(
                xp_hbm.at[pl.ds(base + S_LO * C, S_HI * C)], idx_hi[buf],
                ssems[buf])
            return lo, hi

        def gather_start(buf):
            lo = pltpu.async_copy(v_sh.at[idx_lo[buf]], vals_lo[buf],
                                  gsems_lo[buf])
            hi = pltpu.async_copy(v_hbm.at[idx_hi[buf]], vals_hi[buf],
                                  gsems_hi[buf])
            return lo, hi

        s_first = stage_start(0, 0)

        @pl.when(sid == 0)
        def _copy_v():
            pltpu.sync_copy(v_hbm, v_sh)

        plsc.subcore_barrier()

        s_first[0].wait()
        s_first[1].wait()
        s_next = stage_start(1, 1) if n_chunks > 1 else None
        g_cur = gather_start(0)

        for ci in range(n_chunks):
            buf = ci % 2
            nbuf = 1 - buf
            if ci + 1 < n_chunks:
                s_next[0].wait()
                s_next[1].wait()
                g_next = gather_start(nbuf)
            g_cur[0].wait()
            g_cur[1].wait()
            if ci + 2 < n_chunks:
                s_next = stage_start(ci + 2, buf)

            def lo_body(si, acc, _vr=vals_lo[buf]):
                return tuple(
                    acc[jj] + _vr[pl.ds(si * C + jj * 16, 16)]
                    for jj in range(n_seg)
                )

            def hi_body(si, acc, _vr=vals_hi[buf]):
                return tuple(
                    acc[jj] + _vr[pl.ds(si * C + jj * 16, 16)]
                    for jj in range(n_seg)
                )

            acc0 = tuple(jnp.zeros((16,), jnp.float32) for _ in range(n_seg))
            acc = lax.fori_loop(0, S_LO, lo_body, acc0)
            acc = lax.fori_loop(0, S_HI, hi_body, acc)

            for jj in range(n_seg):
                z = acc[jj]
                outb_v[pl.ds(jj * 16, 16)] = 1.0 / (1.0 + jnp.exp(-z))
            base = wid * per_w + ci * C
            pltpu.sync_copy(outb_v, out_hbm.at[pl.ds(base, C)])
            if ci + 1 < n_chunks:
                g_cur = g_next

    return sc_gather


def kernel(x, table, W, b):
    B, S = x.shape
    V, D = table.shape

    x = x.astype(jnp.int32)

    info = plsc.get_sparse_core_info()
    NC, NS = info.num_cores, info.num_subcores
    NW = NC * NS
    C = 64
    n_chunks = B // (NW * C)
    xp = x.reshape(NW, n_chunks, C, S).swapaxes(2, 3).reshape(B * S)

    w_scaled = (W * (1.0 / S)).astype(jnp.float32)
    bias_scaled = (b * (1.0 / S)).reshape(1, 1).astype(jnp.float32)

    tT = jnp.swapaxes(table, 0, 1)
    v = _project_table(tT, w_scaled, bias_scaled)

    sc_gather = _make_sc_gather(V, S, B, NC, NS, C)
    return sc_gather(v, xp)


# speedup vs baseline: 1.1484x; 1.1484x over previous
import functools

import jax
import jax.numpy as jnp
from jax import lax
from jax.experimental import pallas as pl
from jax.experimental.pallas import tpu as pltpu
from jax.experimental.pallas import tpu_sc as plsc


def _proj_body(tblT_ref, w_ref, bias_ref, out_ref):
    r = lax.dot_general(
        w_ref[...], tblT_ref[...],
        (((1,), (0,)), ((), ())),
        preferred_element_type=jnp.float32,
    )
    out_ref[...] = r[0] + bias_ref[0, 0]


def _project_table(tableT, w_scaled, bias_scaled, blk=65536):
    D, V = tableT.shape
    grid = pl.cdiv(V, blk)
    return pl.pallas_call(
        _proj_body,
        grid=(grid,),
        in_specs=[
            pl.BlockSpec((D, blk), lambda i: (0, i)),
            pl.BlockSpec((1, D), lambda i: (0, 0)),
            pl.BlockSpec(memory_space=pltpu.SMEM),
        ],
        out_specs=pl.BlockSpec((blk,), lambda i: (i,)),
        out_shape=jax.ShapeDtypeStruct((V,), jnp.float32),
    )(tableT, w_scaled, bias_scaled)


def _make_sc_gather(V, S, B, NC, NS, C):
    NW = NC * NS
    per_w = B // NW
    n_chunks = per_w // C
    n_seg = C // 16
    S_LO = S // 2
    S_HI = S - S_LO

    mesh = plsc.VectorSubcoreMesh(core_axis_name="c", subcore_axis_name="s")

    @functools.partial(
        pl.kernel,
        mesh=mesh,
        out_type=jax.ShapeDtypeStruct((B,), jnp.float32),
        scratch_types=[
            pltpu.VMEM((S_LO * C,), jnp.int32),
            pltpu.VMEM((S_HI * C,), jnp.int32),
            pltpu.VMEM((S_LO * C,), jnp.int32),
            pltpu.VMEM((S_HI * C,), jnp.int32),
            pltpu.VMEM((S_LO * C,), jnp.float32),
            pltpu.VMEM((S_HI * C,), jnp.float32),
            pltpu.VMEM((S_LO * C,), jnp.float32),
            pltpu.VMEM((S_HI * C,), jnp.float32),
            pltpu.VMEM((C,), jnp.float32),
            pltpu.VMEM_SHARED((V,), jnp.float32),
            pltpu.SemaphoreType.DMA,
            pltpu.SemaphoreType.DMA,
            pltpu.SemaphoreType.DMA,
            pltpu.SemaphoreType.DMA,
        ],
    )
    def sc_gather(v_hbm, xp_hbm, out_hbm,
                  il0, ih0, il1, ih1, vl0, vh0, vl1, vh1,
                  outb_v, v_sh, ssem0, ssem1, gsem_lo, gsem_hi):
        sid = lax.axis_index("s")
        wid = sid * NC + lax.axis_index("c")
        idx_lo = (il0, il1)
        idx_hi = (ih0, ih1)
        vals_lo = (vl0, vl1)
        vals_hi = (vh0, vh1)
        ssems = (ssem0, ssem1)

        def stage_start(ci, buf):
            base = (wid * per_w + ci * C) * S
            lo = pltpu.async_copy(
                xp_hbm.at[pl.ds(base, S_LO * C)], idx_lo[buf], ssems[buf])
            hi = pltpu.async_copy(
                xp_hbm.at[pl.ds(base + S_LO * C, S_HI * C)], idx_hi[buf],
                ssems[buf])
            return lo, hi

        def gather_start(buf):
            lo = pltpu.async_copy(v_sh.at[idx_lo[buf]], vals_lo[buf], gsem_lo)
            hi = pltpu.async_copy(v_sh.at[idx_hi[buf]], vals_hi[buf], gsem_hi)
            return lo, hi

        s_first = stage_start(0, 0)

        @pl.when(sid == 0)
        def _copy_v():
            pltpu.sync_copy(v_hbm, v_sh)

        plsc.subcore_barrier()

        s_first[0].wait()
        s_first[1].wait()
        s_next = stage_start(1, 1) if n_chunks > 1 else None
        g_cur = gather_start(0)

        for ci in range(n_chunks):
            buf = ci % 2
            nbuf = 1 - buf
            if ci + 1 < n_chunks:
                s_next[0].wait()
                s_next[1].wait()
                g_next = gather_start(nbuf)
            g_cur[0].wait()
            g_cur[1].wait()
            if ci + 2 < n_chunks:
                s_next = stage_start(ci + 2, buf)

            def lo_body(si, acc, _vr=vals_lo[buf]):
                return tuple(
                    acc[jj] + _vr[pl.ds(si * C + jj * 16, 16)]
                    for jj in range(n_seg)
                )

            def hi_body(si, acc, _vr=vals_hi[buf]):
                return tuple(
                    acc[jj] + _vr[pl.ds(si * C + jj * 16, 16)]
                    for jj in range(n_seg)
                )

            acc0 = tuple(jnp.zeros((16,), jnp.float32) for _ in range(n_seg))
            acc = lax.fori_loop(0, S_LO, lo_body, acc0)
            acc = lax.fori_loop(0, S_HI, hi_body, acc)

            for jj in range(n_seg):
                z = acc[jj]
                outb_v[pl.ds(jj * 16, 16)] = 1.0 / (1.0 + jnp.exp(-z))
            base = wid * per_w + ci * C
            pltpu.sync_copy(outb_v, out_hbm.at[pl.ds(base, C)])
            if ci + 1 < n_chunks:
                g_cur = g_next

    return sc_gather


def kernel(x, table, W, b):
    B, S = x.shape
    V, D = table.shape

    x = x.astype(jnp.int32)

    info = plsc.get_sparse_core_info()
    NC, NS = info.num_cores, info.num_subcores
    NW = NC * NS
    C = 64
    n_chunks = B // (NW * C)
    xp = x.reshape(NW, n_chunks, C, S).swapaxes(2, 3).reshape(B * S)

    w_scaled = (W * (1.0 / S)).astype(jnp.float32)
    bias_scaled = (b * (1.0 / S)).reshape(1, 1).astype(jnp.float32)

    tT = jnp.swapaxes(table, 0, 1)
    v = _project_table(tT, w_scaled, bias_scaled)

    sc_gather = _make_sc_gather(V, S, B, NC, NS, C)
    return sc_gather(v, xp)


# speedup vs baseline: 1.2077x; 1.0517x over previous
import functools

import jax
import jax.numpy as jnp
from jax import lax
from jax.experimental import pallas as pl
from jax.experimental.pallas import tpu as pltpu
from jax.experimental.pallas import tpu_sc as plsc


def _proj_body(tblT_ref, w_ref, bias_ref, out_ref):
    r = lax.dot_general(
        w_ref[...], tblT_ref[...],
        (((1,), (0,)), ((), ())),
        preferred_element_type=jnp.float32,
    )
    out_ref[...] = r[0] + bias_ref[0, 0]


def _project_table(tableT, w_scaled, bias_scaled, blk=65536):
    D, V = tableT.shape
    grid = pl.cdiv(V, blk)
    return pl.pallas_call(
        _proj_body,
        grid=(grid,),
        in_specs=[
            pl.BlockSpec((D, blk), lambda i: (0, i)),
            pl.BlockSpec((1, D), lambda i: (0, 0)),
            pl.BlockSpec(memory_space=pltpu.SMEM),
        ],
        out_specs=pl.BlockSpec((blk,), lambda i: (i,)),
        out_shape=jax.ShapeDtypeStruct((V,), jnp.float32),
    )(tableT, w_scaled, bias_scaled)


def _make_sc_gather(V, S, B, NC, NS, C):
    NW = NC * NS
    per_w = B // NW
    n_chunks = per_w // C
    n_full = S // 16
    tail = S - n_full * 16

    mesh = plsc.VectorSubcoreMesh(core_axis_name="c", subcore_axis_name="s")

    @functools.partial(
        pl.kernel,
        mesh=mesh,
        compiler_params=pltpu.CompilerParams(needs_layout_passes=False),
        out_type=jax.ShapeDtypeStruct((B,), jnp.float32),
        scratch_types=[
            pltpu.VMEM((S * C,), jnp.int32),
            pltpu.VMEM((S * C,), jnp.int32),
            pltpu.VMEM((S * C,), jnp.float32),
            pltpu.VMEM((S * C,), jnp.float32),
            pltpu.VMEM((C,), jnp.float32),
            pltpu.VMEM_SHARED((V,), jnp.float32),
            pltpu.SemaphoreType.DMA,
            pltpu.SemaphoreType.DMA,
            pltpu.SemaphoreType.DMA,
            pltpu.SemaphoreType.DMA,
        ],
    )
    def sc_gather(v_hbm, xf_hbm, out_hbm, idx0_v, idx1_v, vals0_v, vals1_v,
                  outb_v, v_sh, ssem0, ssem1, gsem0, gsem1):
        sid = lax.axis_index("s")
        wid = sid * NC + lax.axis_index("c")
        idxs = (idx0_v, idx1_v)
        vals = (vals0_v, vals1_v)
        ssems = (ssem0, ssem1)
        gsems = (gsem0, gsem1)

        def stage_start(ci, buf):
            row0 = (wid * n_chunks + ci) * C
            return pltpu.async_copy(
                xf_hbm.at[pl.ds(row0 * S, S * C)], idxs[buf], ssems[buf])

        def gather_start(buf):
            return pltpu.async_copy(
                v_sh.at[idxs[buf]], vals[buf], gsems[buf])

        s_first = stage_start(0, 0)

        @pl.when(sid == 0)
        def _copy_v():
            pltpu.sync_copy(v_hbm, v_sh)

        plsc.subcore_barrier()

        s_first.wait()
        s_next = stage_start(1, 1) if n_chunks > 1 else None
        g_cur = gather_start(0)

        lane = lax.iota(jnp.int32, 16)
        tail_keep = lane >= (16 - tail)
        last_lane = jnp.full((16,), 15, jnp.int32)

        for ci in range(n_chunks):
            buf = ci % 2
            nbuf = 1 - buf
            if ci + 1 < n_chunks:
                s_next.wait()
                g_next = gather_start(nbuf)
            g_cur.wait()
            if ci + 2 < n_chunks:
                s_next = stage_start(ci + 2, buf)

            def group_body(g, carry, _vr=vals[buf]):
                off0 = (g * 16 + lane) * S

                def s_body(si, acc):
                    return acc + plsc.load_gather(_vr, [off0 + si])

                res = lax.fori_loop(
                    0, S, s_body, jnp.zeros((16,), jnp.float32))
                outb_v[pl.ds(g * 16, 16)] = 1.0 / (1.0 + jnp.exp(-res))
                return carry

            lax.fori_loop(0, C // 16, group_body, 0)

            base = (wid * n_chunks + ci) * C
            pltpu.sync_copy(outb_v, out_hbm.at[pl.ds(base, C)])
            if ci + 1 < n_chunks:
                g_cur = g_next

    return sc_gather


def kernel(x, table, W, b):
    B, S = x.shape
    V, D = table.shape

    xf = x.astype(jnp.int32).reshape(B * S)

    info = plsc.get_sparse_core_info()
    NC, NS = info.num_cores, info.num_subcores

    w_scaled = (W * (1.0 / S)).astype(jnp.float32)
    bias_scaled = (b * (1.0 / S)).reshape(1, 1).astype(jnp.float32)

    tT = jnp.swapaxes(table, 0, 1)
    v = _project_table(tT, w_scaled, bias_scaled)

    sc_gather = _make_sc_gather(V, S, B, NC, NS, C=64)
    return sc_gather(v, xf)


# speedup vs baseline: 1.2603x; 1.0436x over previous
import functools

import jax
import jax.numpy as jnp
from jax import lax
from jax.experimental import pallas as pl
from jax.experimental.pallas import tpu as pltpu
from jax.experimental.pallas import tpu_sc as plsc


def _proj_body(tblT_ref, w_ref, bias_ref, out_ref):
    r = lax.dot_general(
        w_ref[...], tblT_ref[...],
        (((1,), (0,)), ((), ())),
        preferred_element_type=jnp.float32,
    )
    out_ref[...] = r[0] + bias_ref[0, 0]


def _project_table(tableT, w_scaled, bias_scaled, blk=65536):
    D, V = tableT.shape
    grid = pl.cdiv(V, blk)
    return pl.pallas_call(
        _proj_body,
        grid=(grid,),
        in_specs=[
            pl.BlockSpec((D, blk), lambda i: (0, i)),
            pl.BlockSpec((1, D), lambda i: (0, 0)),
            pl.BlockSpec(memory_space=pltpu.SMEM),
        ],
        out_specs=pl.BlockSpec((blk,), lambda i: (i,)),
        out_shape=jax.ShapeDtypeStruct((V,), jnp.float32),
    )(tableT, w_scaled, bias_scaled)


def _make_sc_gather(V, S, B, NC, NS, C):
    NW = NC * NS
    per_w = B // NW
    n_chunks = per_w // C
    n_full = S // 16
    tail = S - n_full * 16

    mesh = plsc.VectorSubcoreMesh(core_axis_name="c", subcore_axis_name="s")

    @functools.partial(
        pl.kernel,
        mesh=mesh,
        compiler_params=pltpu.CompilerParams(needs_layout_passes=False),
        out_type=jax.ShapeDtypeStruct((B,), jnp.float32),
        scratch_types=[
            pltpu.VMEM((S * C,), jnp.int32),
            pltpu.VMEM((S * C,), jnp.int32),
            pltpu.VMEM((S * C,), jnp.float32),
            pltpu.VMEM((S * C,), jnp.float32),
            pltpu.VMEM((C,), jnp.float32),
            pltpu.VMEM_SHARED((V,), jnp.float32),
            pltpu.SemaphoreType.DMA,
            pltpu.SemaphoreType.DMA,
            pltpu.SemaphoreType.DMA,
            pltpu.SemaphoreType.DMA,
        ],
    )
    def sc_gather(v_hbm, xf_hbm, out_hbm, idx0_v, idx1_v, vals0_v, vals1_v,
                  outb_v, v_sh, ssem0, ssem1, gsem0, gsem1):
        sid = lax.axis_index("s")
        wid = sid * NC + lax.axis_index("c")
        idxs = (idx0_v, idx1_v)
        vals = (vals0_v, vals1_v)
        ssems = (ssem0, ssem1)
        gsems = (gsem0, gsem1)

        def stage_start(ci, buf):
            row0 = (wid * n_chunks + ci) * C
            return pltpu.async_copy(
                xf_hbm.at[pl.ds(row0 * S, S * C)], idxs[buf], ssems[buf])

        def gather_start(buf):
            return pltpu.async_copy(
                v_sh.at[idxs[buf]], vals[buf], gsems[buf])

        s_first = stage_start(0, 0)

        @pl.when(sid == 0)
        def _copy_v():
            pltpu.sync_copy(v_hbm, v_sh)

        plsc.subcore_barrier()

        s_first.wait()
        s_next = stage_start(1, 1) if n_chunks > 1 else None
        g_cur = gather_start(0)

        lane = lax.iota(jnp.int32, 16)
        tail_keep = lane >= (16 - tail)
        last_lane = jnp.full((16,), 15, jnp.int32)

        for ci in range(n_chunks):
            buf = ci % 2
            nbuf = 1 - buf
            if ci + 1 < n_chunks:
                s_next.wait()
                g_next = gather_start(nbuf)
            g_cur.wait()
            if ci + 2 < n_chunks:
                s_next = stage_start(ci + 2, buf)

            def group_body(g, carry, _vr=vals[buf]):
                off0 = (g * 16 + lane) * S

                def s_body(si, accs):
                    s4 = si * 8
                    return tuple(
                        accs[u] + plsc.load_gather(_vr, [off0 + (s4 + u)])
                        for u in range(8)
                    )

                accs = lax.fori_loop(
                    0, S // 8, s_body,
                    tuple(jnp.zeros((16,), jnp.float32) for _ in range(8)))
                res = ((accs[0] + accs[1]) + (accs[2] + accs[3])) + (
                    (accs[4] + accs[5]) + (accs[6] + accs[7]))
                outb_v[pl.ds(g * 16, 16)] = 1.0 / (1.0 + jnp.exp(-res))
                return carry

            lax.fori_loop(0, C // 16, group_body, 0)

            base = (wid * n_chunks + ci) * C
            pltpu.sync_copy(outb_v, out_hbm.at[pl.ds(base, C)])
            if ci + 1 < n_chunks:
                g_cur = g_next

    return sc_gather


def kernel(x, table, W, b):
    B, S = x.shape
    V, D = table.shape

    xf = x.astype(jnp.int32).reshape(B * S)

    info = plsc.get_sparse_core_info()
    NC, NS = info.num_cores, info.num_subcores

    w_scaled = (W * (1.0 / S)).astype(jnp.float32)
    bias_scaled = (b * (1.0 / S)).reshape(1, 1).astype(jnp.float32)

    tT = jnp.swapaxes(table, 0, 1)
    v = _project_table(tT, w_scaled, bias_scaled)

    sc_gather = _make_sc_gather(V, S, B, NC, NS, C=64)
    return sc_gather(v, xf)


# speedup vs baseline: 1.3411x; 1.0641x over previous
import functools

import jax
import jax.numpy as jnp
from jax import lax
from jax.experimental import pallas as pl
from jax.experimental.pallas import tpu as pltpu
from jax.experimental.pallas import tpu_sc as plsc


def _proj_body(tblT_ref, w_ref, bias_ref, out_ref):
    r = lax.dot_general(
        w_ref[...], tblT_ref[...],
        (((1,), (0,)), ((), ())),
        preferred_element_type=jnp.float32,
    )
    out_ref[...] = r[0] + bias_ref[0, 0]


def _project_table(tableT, w_scaled, bias_scaled, blk=65536):
    D, V = tableT.shape
    grid = pl.cdiv(V, blk)
    return pl.pallas_call(
        _proj_body,
        grid=(grid,),
        in_specs=[
            pl.BlockSpec((D, blk), lambda i: (0, i)),
            pl.BlockSpec((1, D), lambda i: (0, 0)),
            pl.BlockSpec(memory_space=pltpu.SMEM),
        ],
        out_specs=pl.BlockSpec((blk,), lambda i: (i,)),
        out_shape=jax.ShapeDtypeStruct((V,), jnp.float32),
    )(tableT, w_scaled, bias_scaled)


def _make_sc_gather(V, S, B, NC, NS, C):
    NW = NC * NS
    per_w = B // NW
    n_chunks = per_w // C
    n_full = S // 16
    tail = S - n_full * 16

    mesh = plsc.VectorSubcoreMesh(core_axis_name="c", subcore_axis_name="s")

    @functools.partial(
        pl.kernel,
        mesh=mesh,
        compiler_params=pltpu.CompilerParams(needs_layout_passes=False),
        out_type=jax.ShapeDtypeStruct((B,), jnp.float32),
        scratch_types=[
            pltpu.VMEM((C, S), jnp.int32),
            pltpu.VMEM((S * C,), jnp.int32),
            pltpu.VMEM((S * C,), jnp.int32),
            pltpu.VMEM((S * C,), jnp.float32),
            pltpu.VMEM((S * C,), jnp.float32),
            pltpu.VMEM((C,), jnp.float32),
            pltpu.VMEM_SHARED((V,), jnp.float32),
            pltpu.SemaphoreType.DMA,
            pltpu.SemaphoreType.DMA,
            pltpu.SemaphoreType.DMA,
            pltpu.SemaphoreType.DMA,
        ],
    )
    def sc_gather(v_hbm, xf_hbm, out_hbm, idx2d_v, idx0_v, idx1_v,
                  vals0_v, vals1_v, outb_v, v_sh, ssem0, ssem1, gsem0, gsem1):
        sid = lax.axis_index("s")
        wid = sid * NC + lax.axis_index("c")
        idxs = (idx0_v, idx1_v)
        vals = (vals0_v, vals1_v)
        ssems = (ssem0, ssem1)
        gsems = (gsem0, gsem1)

        lane = lax.iota(jnp.int32, 16)

        def stage_start(ci, buf):
            row0 = (wid * n_chunks + ci) * C
            return pltpu.async_copy(
                xf_hbm.at[pl.ds(row0, C), :], idx2d_v, ssems[buf])

        def relayout(buf):
            def row(j, carry, _dst=idxs[buf]):
                jv = jnp.zeros((16,), jnp.int32) + j
                for k in range(S // 16):
                    sv = k * 16 + lane
                    _dst[pl.ds(j * S + k * 16, 16)] = plsc.load_gather(
                        idx2d_v, [jv, sv])
                if S % 16:
                    sv = (S - 16) + lane
                    _dst[pl.ds(j * S + S - 16, 16)] = plsc.load_gather(
                        idx2d_v, [jv, sv])
                return carry

            lax.fori_loop(0, C, row, 0)

        def gather_start(buf):
            return pltpu.async_copy(
                v_sh.at[idxs[buf]], vals[buf], gsems[buf])

        s_first = stage_start(0, 0)

        @pl.when(sid == 0)
        def _copy_v():
            pltpu.sync_copy(v_hbm, v_sh)

        plsc.subcore_barrier()

        s_first.wait()
        relayout(0)
        g_cur = gather_start(0)
        s_next = stage_start(1, 1) if n_chunks > 1 else None
        tail_keep = lane >= (16 - tail)
        last_lane = jnp.full((16,), 15, jnp.int32)

        for ci in range(n_chunks):
            buf = ci % 2
            nbuf = 1 - buf
            if ci + 1 < n_chunks:
                s_next.wait()
                relayout(nbuf)
                g_next = gather_start(nbuf)
                if ci + 2 < n_chunks:
                    s_next = stage_start(ci + 2, nbuf)
            g_cur.wait()

            def group_body(g, carry, _vr=vals[buf]):
                off0 = (g * 16 + lane) * S

                def s_body(si, accs):
                    s4 = si * 8
                    return tuple(
                        accs[u] + plsc.load_gather(_vr, [off0 + (s4 + u)])
                        for u in range(8)
                    )

                accs = lax.fori_loop(
                    0, S // 8, s_body,
                    tuple(jnp.zeros((16,), jnp.float32) for _ in range(8)))
                res = ((accs[0] + accs[1]) + (accs[2] + accs[3])) + (
                    (accs[4] + accs[5]) + (accs[6] + accs[7]))
                outb_v[pl.ds(g * 16, 16)] = 1.0 / (1.0 + jnp.exp(-res))
                return carry

            lax.fori_loop(0, C // 16, group_body, 0)

            base = (wid * n_chunks + ci) * C
            pltpu.sync_copy(outb_v, out_hbm.at[pl.ds(base, C)])
            if ci + 1 < n_chunks:
                g_cur = g_next

    return sc_gather


def kernel(x, table, W, b):
    B, S = x.shape
    V, D = table.shape

    xf = x.astype(jnp.int32)

    info = plsc.get_sparse_core_info()
    NC, NS = info.num_cores, info.num_subcores

    w_scaled = (W * (1.0 / S)).astype(jnp.float32)
    bias_scaled = (b * (1.0 / S)).reshape(1, 1).astype(jnp.float32)

    tT = jnp.swapaxes(table, 0, 1)
    v = _project_table(tT, w_scaled, bias_scaled)

    sc_gather = _make_sc_gather(V, S, B, NC, NS, C=64)
    return sc_gather(v, xf)
